# baseline (speedup 1.0000x reference)
import functools

import jax
import jax.numpy as jnp
from jax import lax
from jax.experimental import pallas as pl
from jax.experimental.pallas import tpu as pltpu
from jax.experimental.pallas import tpu_sc as plsc

VOCAB = 1000000
D = 64
W = 200
B = 4096

NC = 2
NS = 16
NW = NC * NS

ROWS = B * W
ROWS_PER_W = ROWS // NW
CHUNKS_PER_W = ROWS_PER_W // W


def _sc_body(idx_hbm, table_hbm, pe_hbm, out_hbm, idx_v, rows_v, pe_v, sem):
    wid = lax.axis_index("s") * NC + lax.axis_index("c")
    base_w = wid * ROWS_PER_W

    pltpu.sync_copy(pe_hbm, pe_v)

    @pl.loop(0, CHUNKS_PER_W)
    def _chunk(g):
        base = base_w + g * W

        pltpu.sync_copy(idx_hbm.at[pl.ds(base, W)], idx_v)

        cp0 = pltpu.async_copy(
            table_hbm.at[idx_v.at[pl.ds(0, 128)]], rows_v.at[pl.ds(0, 128)], sem)
        cp1 = pltpu.async_copy(
            table_hbm.at[idx_v.at[pl.ds(128, W - 128)]],
            rows_v.at[pl.ds(128, W - 128)], sem)
        cp0.wait()
        cp1.wait()

        @pl.loop(0, W)
        def _row(w):
            for c in range(D // 16):
                s = pl.ds(c * 16, 16)
                rows_v[w, s] = rows_v[w, s] + pe_v[w, s]

        pltpu.sync_copy(rows_v, out_hbm.at[pl.ds(base, W)])


def kernel(x, table, pos_enc):
    idx = x.reshape(ROWS).astype(jnp.int32)
    mesh = plsc.VectorSubcoreMesh(core_axis_name="c", subcore_axis_name="s")
    out = pl.kernel(
        _sc_body,
        out_type=jax.ShapeDtypeStruct((ROWS, D), jnp.float32),
        mesh=mesh,
        compiler_params=pltpu.CompilerParams(use_tc_tiling_on_sc=False),
        scratch_types=[
            pltpu.VMEM((W,), jnp.int32),
            pltpu.VMEM((W, D), jnp.float32),
            pltpu.VMEM((W, D), jnp.float32),
            pltpu.SemaphoreType.DMA,
        ],
    )(idx, table, pos_enc)
    return out.reshape(B, W, D)


# speedup vs baseline: 1.1289x; 1.1289x over previous
import jax
import jax.numpy as jnp
from jax import lax
from jax.experimental import pallas as pl
from jax.experimental.pallas import tpu as pltpu
from jax.experimental.pallas import tpu_sc as plsc

VOCAB = 1000000
D = 64
W = 200
B = 4096

NC = 2
NS = 16
NW = NC * NS

ROWS = B * W
ROWS_PER_W = ROWS // NW
CHUNKS_PER_W = ROWS_PER_W // W


def _gather(table_hbm, idx_v, b, rows_v, sem):
    c0 = pltpu.async_copy(
        table_hbm.at[idx_v.at[b, pl.ds(0, 128)]],
        rows_v.at[b, pl.ds(0, 128)], sem)
    c1 = pltpu.async_copy(
        table_hbm.at[idx_v.at[b, pl.ds(128, W - 128)]],
        rows_v.at[b, pl.ds(128, W - 128)], sem)
    return c0, c1


def _sc_body(idx_hbm, table_hbm, pe_hbm, out_hbm,
             idx_v, rows_v, pe_v, gsem, osem):
    wid = lax.axis_index("s") * NC + lax.axis_index("c")
    base_w = wid * ROWS_PER_W

    pltpu.sync_copy(pe_hbm, pe_v)

    pltpu.sync_copy(idx_hbm.at[pl.ds(base_w, W)], idx_v.at[0])
    _gather(table_hbm, idx_v, 0, rows_v, gsem.at[0])

    @pl.loop(0, CHUNKS_PER_W, step=2)
    def _pair(g0):
        for b in range(2):
            g = g0 + b
            nb = 1 - b
            base = base_w + g * W

            @pl.when(g + 1 < CHUNKS_PER_W)
            def _prefetch():
                @pl.when(g >= 1)
                def _drain_prev_store():
                    pltpu.make_async_copy(
                        rows_v.at[nb], out_hbm.at[pl.ds(base - W, W)],
                        osem.at[nb]).wait()

                pltpu.sync_copy(
                    idx_hbm.at[pl.ds(base + W, W)], idx_v.at[nb])
                _gather(table_hbm, idx_v, nb, rows_v, gsem.at[nb])

            c0 = pltpu.make_async_copy(
                table_hbm.at[idx_v.at[b, pl.ds(0, 128)]],
                rows_v.at[b, pl.ds(0, 128)], gsem.at[b])
            c1 = pltpu.make_async_copy(
                table_hbm.at[idx_v.at[b, pl.ds(128, W - 128)]],
                rows_v.at[b, pl.ds(128, W - 128)], gsem.at[b])
            c0.wait()
            c1.wait()

            @pl.loop(0, W)
            def _row(w):
                for c in range(D // 16):
                    s = pl.ds(c * 16, 16)
                    rows_v[b, w, s] = rows_v[b, w, s] + pe_v[w, s]

            pltpu.async_copy(rows_v.at[b], out_hbm.at[pl.ds(base, W)],
                             osem.at[b])

    lastb = (CHUNKS_PER_W - 1) % 2
    pltpu.make_async_copy(
        rows_v.at[lastb],
        out_hbm.at[pl.ds(base_w + (CHUNKS_PER_W - 1) * W, W)],
        osem.at[lastb]).wait()


def kernel(x, table, pos_enc):
    idx = x.reshape(ROWS).astype(jnp.int32)
    mesh = plsc.VectorSubcoreMesh(core_axis_name="c", subcore_axis_name="s")
    out = pl.kernel(
        _sc_body,
        out_type=jax.ShapeDtypeStruct((ROWS, D), jnp.float32),
        mesh=mesh,
        compiler_params=pltpu.CompilerParams(use_tc_tiling_on_sc=False),
        scratch_types=[
            pltpu.VMEM((2, W), jnp.int32),
            pltpu.VMEM((2, W, D), jnp.float32),
            pltpu.VMEM((W, D), jnp.float32),
            pltpu.SemaphoreType.DMA((2,)),
            pltpu.SemaphoreType.DMA((2,)),
        ],
    )(idx, table, pos_enc)
    return out.reshape(B, W, D)
